# baseline (speedup 1.0000x reference)
import jax
import jax.numpy as jnp
from jax.experimental import pallas as pl

BATCH = 4096
SEQ = 200
H_DIM = 32
BB = 64


def _bcast_body(emb_ref, out_ref):
    out_ref[...] = jnp.broadcast_to(emb_ref[...][None], out_ref.shape)


def kernel(x, pos_embedding):
    del x
    return pl.pallas_call(
        _bcast_body,
        grid=(BATCH // BB,),
        in_specs=[pl.BlockSpec((SEQ, H_DIM), lambda i: (0, 0))],
        out_specs=pl.BlockSpec((BB, SEQ, H_DIM), lambda i: (i, 0, 0)),
        out_shape=jax.ShapeDtypeStruct((BATCH, SEQ, H_DIM), jnp.float32),
    )(pos_embedding[:SEQ])


# speedup vs baseline: 5.4927x; 5.4927x over previous
import functools
import jax
import jax.numpy as jnp
from jax import lax
from jax.experimental import pallas as pl
from jax.experimental.pallas import tpu as pltpu
from jax.experimental.pallas import tpu_sc as plsc

BATCH = 4096
SEQ = 200
H_DIM = 32
ROWS = SEQ * H_DIM

_INFO = plsc.get_sparse_core_info()
NC, NS = _INFO.num_cores, _INFO.num_subcores
NW = NC * NS
RPW = ROWS // NW
CH = 8
NCH = RPW // CH
VPR = BATCH // 16


def _sc_body(splat_hbm, out_hbm, splat_v, buf0, buf1, sem0, sem1):
    wid = lax.axis_index("s") * NC + lax.axis_index("c")
    base = wid * RPW
    pltpu.sync_copy(splat_hbm.at[pl.ds(base * 16, RPW * 16)], splat_v)
    bufs = (buf0, buf1)
    sems = (sem0, sem1)

    def fill(buf, c):
        def kbody(k, carry):
            off = k * 128
            for i in range(CH):
                sp = splat_v[pl.ds((c * CH + i) * 16, 16)]
                for j in range(8):
                    buf[i, pl.ds(off + j * 16, 16)] = sp
            return carry

        lax.fori_loop(0, VPR // 8, kbody, 0)

    def start_dma(buf, sem, c):
        pltpu.async_copy(buf, out_hbm.at[pl.ds(base + c * CH, CH)], sem)

    def wait_dma(buf, sem):
        pltpu.make_async_copy(buf, out_hbm.at[pl.ds(base, CH)], sem).wait()

    def pair_body(g, carry):
        for b in range(2):
            c = 2 * g + b

            @pl.when(g > 0)
            def _():
                wait_dma(bufs[b], sems[b])

            fill(bufs[b], c)
            start_dma(bufs[b], sems[b], c)
        return carry

    lax.fori_loop(0, NCH // 2, pair_body, 0)
    wait_dma(buf0, sem0)
    fill(buf0, NCH - 1)
    start_dma(buf0, sem0, NCH - 1)
    wait_dma(buf0, sem0)
    wait_dma(buf1, sem1)


_sc_call = functools.partial(
    pl.kernel,
    mesh=plsc.VectorSubcoreMesh(core_axis_name="c", subcore_axis_name="s"),
    out_type=jax.ShapeDtypeStruct((ROWS, BATCH), jnp.float32),
    scratch_types=[
        pltpu.VMEM((RPW * 16,), jnp.float32),
        pltpu.VMEM((CH, BATCH), jnp.float32),
        pltpu.VMEM((CH, BATCH), jnp.float32),
        pltpu.SemaphoreType.DMA,
        pltpu.SemaphoreType.DMA,
    ],
)(_sc_body)


def kernel(x, pos_embedding):
    del x
    splats = jnp.repeat(pos_embedding[:SEQ].reshape(ROWS), 16)
    out = _sc_call(splats)
    return out.reshape(SEQ, H_DIM, BATCH).transpose(2, 0, 1)


# speedup vs baseline: 5.5264x; 1.0061x over previous
import functools
import jax
import jax.numpy as jnp
from jax import lax
from jax.experimental import pallas as pl
from jax.experimental.pallas import tpu as pltpu
from jax.experimental.pallas import tpu_sc as plsc

BATCH = 4096
SEQ = 200
H_DIM = 32
ROWS = SEQ * H_DIM

_INFO = plsc.get_sparse_core_info()
NC, NS = _INFO.num_cores, _INFO.num_subcores
NW = NC * NS
RPW = ROWS // NW
NG = (RPW + 15) // 16
CH = 8
NCH = RPW // CH
VPR = BATCH // 16


def _sc_body(tab_hbm, out_hbm, tab_v, splat_v, buf0, buf1, sem0, sem1):
    wid = lax.axis_index("s") * NC + lax.axis_index("c")
    base = wid * RPW
    pltpu.sync_copy(tab_hbm.at[pl.ds(base, RPW)], tab_v.at[pl.ds(0, RPW)])
    bufs = (buf0, buf1)
    sems = (sem0, sem1)

    def egroup(g, carry):
        vec = tab_v[pl.ds(g * 16, 16)]
        for m in range(16):
            sp = vec.at[jnp.full((16,), m, jnp.int32)].get(
                mode="promise_in_bounds"
            )
            splat_v[pl.ds(g * 256 + m * 16, 16)] = sp
        return carry

    lax.fori_loop(0, NG, egroup, 0)

    def fill(buf, c):
        def kbody(k, carry):
            off = k * 128
            for i in range(CH):
                sp = splat_v[pl.ds((c * CH + i) * 16, 16)]
                for j in range(8):
                    buf[i, pl.ds(off + j * 16, 16)] = sp
            return carry

        lax.fori_loop(0, VPR // 8, kbody, 0)

    def start_dma(buf, sem, c):
        pltpu.async_copy(buf, out_hbm.at[pl.ds(base + c * CH, CH)], sem)

    def wait_dma(buf, sem):
        pltpu.make_async_copy(buf, out_hbm.at[pl.ds(base, CH)], sem).wait()

    def pair_body(g, carry):
        for b in range(2):
            c = 2 * g + b

            @pl.when(g > 0)
            def _():
                wait_dma(bufs[b], sems[b])

            fill(bufs[b], c)
            start_dma(bufs[b], sems[b], c)
        return carry

    lax.fori_loop(0, NCH // 2, pair_body, 0)
    wait_dma(buf0, sem0)
    fill(buf0, NCH - 1)
    start_dma(buf0, sem0, NCH - 1)
    wait_dma(buf0, sem0)
    wait_dma(buf1, sem1)


_sc_call = functools.partial(
    pl.kernel,
    mesh=plsc.VectorSubcoreMesh(core_axis_name="c", subcore_axis_name="s"),
    out_type=jax.ShapeDtypeStruct((ROWS, BATCH), jnp.float32),
    scratch_types=[
        pltpu.VMEM((NG * 16,), jnp.float32),
        pltpu.VMEM((NG * 256,), jnp.float32),
        pltpu.VMEM((CH, BATCH), jnp.float32),
        pltpu.VMEM((CH, BATCH), jnp.float32),
        pltpu.SemaphoreType.DMA,
        pltpu.SemaphoreType.DMA,
    ],
)(_sc_body)


def kernel(x, pos_embedding):
    del x
    out = _sc_call(pos_embedding[:SEQ].reshape(ROWS))
    return out.reshape(SEQ, H_DIM, BATCH).transpose(2, 0, 1)
